# Initial kernel scaffold; baseline (speedup 1.0000x reference)
import jax
import jax.numpy as jnp
from jax.experimental import pallas as pl


def kernel(proposals, box_logits, label_logits, box_cos_logits, box_sin_logits):
    raise NotImplementedError("write your pallas kernel here")


# speedup vs baseline: 10.2581x; 10.2581x over previous
import numpy as np
import jax
import jax.numpy as jnp
from jax.experimental import pallas as pl

_N = 5000
_ROWS = 40
_NPAD = _ROWS * 128
_K = 100
_NEG = -1e30
_CLIP = float(np.float32(np.log(1333.0 / 16.0)))


def _nms_body(inp_ref, out_ref):
    px1 = inp_ref[0]
    py1 = inp_ref[1]
    px2 = inp_ref[2]
    py2 = inp_ref[3]
    tx = inp_ref[4] / 10.0
    ty = inp_ref[5] / 10.0
    tw = inp_ref[6] / 5.0
    th = inp_ref[7] / 5.0
    score = inp_ref[8]
    cosl = inp_ref[9]
    sinl = inp_ref[10]

    wa = px2 - px1
    ha = py2 - py1
    xa = (px2 + px1) * 0.5
    ya = (py2 + py1) * 0.5
    wb = jnp.exp(jnp.minimum(tw, _CLIP)) * wa
    hb = jnp.exp(jnp.minimum(th, _CLIP)) * ha
    xb = tx * wa + xa
    yb = ty * ha + ya
    x1 = jnp.clip(xb - wb * 0.5, 0.0, 1024.0)
    y1 = jnp.clip(yb - hb * 0.5, 0.0, 1024.0)
    x2 = jnp.clip(xb + wb * 0.5, 0.0, 1024.0)
    y2 = jnp.clip(yb + hb * 0.5, 0.0, 1024.0)

    area = jnp.maximum(x2 - x1, 0.0) * jnp.maximum(y2 - y1, 0.0)
    idx = (jax.lax.broadcasted_iota(jnp.int32, (_ROWS, 128), 0) * 128
           + jax.lax.broadcasted_iota(jnp.int32, (_ROWS, 128), 1))
    valid0 = (score > 0.05) & ((x2 - x1) * (y2 - y1) > 0.0) & (idx < _N)
    work0 = jnp.where(valid0, score, _NEG)

    sub8 = jax.lax.broadcasted_iota(jnp.int32, (8, 128), 0)
    lane8 = jax.lax.broadcasted_iota(jnp.int32, (8, 128), 1)
    acc0 = jnp.zeros((8, 128), jnp.float32)

    def body(i, state):
        work, acc = state
        m = jnp.max(work)
        best = jnp.min(jnp.where(work == m, idx, jnp.int32(2 ** 30)))
        selv = (idx == best).astype(jnp.float32)
        bx1 = jnp.sum(x1 * selv)
        by1 = jnp.sum(y1 * selv)
        bx2 = jnp.sum(x2 * selv)
        by2 = jnp.sum(y2 * selv)
        bar = jnp.sum(area * selv)
        bco = jnp.sum(cosl * selv)
        bsi = jnp.sum(sinl * selv)
        xx1 = jnp.maximum(x1, bx1)
        yy1 = jnp.maximum(y1, by1)
        xx2 = jnp.minimum(x2, bx2)
        yy2 = jnp.minimum(y2, by2)
        inter = jnp.maximum(xx2 - xx1, 0.0) * jnp.maximum(yy2 - yy1, 0.0)
        union = area + bar - inter
        iou = inter / jnp.maximum(union, 1e-8)
        work = jnp.where(iou >= 0.5, _NEG, work)
        vf = (m > _NEG * 0.5).astype(jnp.float32)
        vals = jnp.where(
            sub8 == 0, bx1,
            jnp.where(sub8 == 1, by1,
                      jnp.where(sub8 == 2, bx2,
                                jnp.where(sub8 == 3, by2,
                                          jnp.where(sub8 == 4, 1.0,
                                                    jnp.where(sub8 == 5, m,
                                                              jnp.where(sub8 == 6, bco, bsi))))))) * vf
        acc = jnp.where(lane8 == i, vals, acc)
        return work, acc

    _, acc = jax.lax.fori_loop(0, _K, body, (work0, acc0))
    out_ref[...] = acc


def kernel(proposals, box_logits, label_logits, box_cos_logits, box_sin_logits):
    def pad(col):
        return jnp.pad(col, (0, _NPAD - _N)).reshape(_ROWS, 128)

    chans = [proposals[:, 0], proposals[:, 1], proposals[:, 2], proposals[:, 3],
             box_logits[:, 0], box_logits[:, 1], box_logits[:, 2], box_logits[:, 3],
             label_logits[:, 1], box_cos_logits, box_sin_logits]
    inp = jnp.stack([pad(c) for c in chans])
    out = pl.pallas_call(
        _nms_body,
        out_shape=jax.ShapeDtypeStruct((8, 128), jnp.float32),
    )(inp)
    return out[:, :_K].T


# speedup vs baseline: 16.0336x; 1.5630x over previous
import numpy as np
import jax
import jax.numpy as jnp
from jax.experimental import pallas as pl
from jax.experimental.pallas import tpu as pltpu

_N = 5000
_ROWS = 40
_NPAD = _ROWS * 128
_K = 100
_NEG = -1e30
_CLIP = float(np.float32(np.log(1333.0 / 16.0)))


def _nms_body(inp_ref, out_ref):
    px1 = inp_ref[0]
    py1 = inp_ref[1]
    px2 = inp_ref[2]
    py2 = inp_ref[3]
    tx = inp_ref[4] / 10.0
    ty = inp_ref[5] / 10.0
    tw = inp_ref[6] / 5.0
    th = inp_ref[7] / 5.0
    score = inp_ref[8]
    cosl = inp_ref[9]
    sinl = inp_ref[10]

    wa = px2 - px1
    ha = py2 - py1
    xa = (px2 + px1) * 0.5
    ya = (py2 + py1) * 0.5
    wb = jnp.exp(jnp.minimum(tw, _CLIP)) * wa
    hb = jnp.exp(jnp.minimum(th, _CLIP)) * ha
    xb = tx * wa + xa
    yb = ty * ha + ya
    x1 = jnp.clip(xb - wb * 0.5, 0.0, 1024.0)
    y1 = jnp.clip(yb - hb * 0.5, 0.0, 1024.0)
    x2 = jnp.clip(xb + wb * 0.5, 0.0, 1024.0)
    y2 = jnp.clip(yb + hb * 0.5, 0.0, 1024.0)

    area = jnp.maximum(x2 - x1, 0.0) * jnp.maximum(y2 - y1, 0.0)
    idx = (jax.lax.broadcasted_iota(jnp.int32, (_ROWS, 128), 0) * 128
           + jax.lax.broadcasted_iota(jnp.int32, (_ROWS, 128), 1))
    valid0 = (score > 0.05) & ((x2 - x1) * (y2 - y1) > 0.0) & (idx < _N)
    work0 = jnp.where(valid0, score, _NEG)

    sub8 = jax.lax.broadcasted_iota(jnp.int32, (8, 128), 0)
    lane8 = jax.lax.broadcasted_iota(jnp.int32, (8, 128), 1)
    acc0 = jnp.zeros((8, 128), jnp.float32)
    fidx = idx.astype(jnp.float32)

    def body(i, state):
        work, acc = state

        bval = jnp.max(work)
        sel0 = work == bval
        z = jnp.float32(0.0)
        cnt = jnp.sum(jnp.where(sel0, 1.0, z))
        g = (jnp.sum(jnp.where(sel0, x1, z)), jnp.sum(jnp.where(sel0, y1, z)),
             jnp.sum(jnp.where(sel0, x2, z)), jnp.sum(jnp.where(sel0, y2, z)),
             jnp.sum(jnp.where(sel0, cosl, z)), jnp.sum(jnp.where(sel0, sinl, z)))

        def tie_fix(_):
            bfi = jnp.min(jnp.where(sel0, fidx, jnp.float32(2 ** 30)))
            sel1 = fidx == bfi
            return (jnp.sum(jnp.where(sel1, x1, z)), jnp.sum(jnp.where(sel1, y1, z)),
                    jnp.sum(jnp.where(sel1, x2, z)), jnp.sum(jnp.where(sel1, y2, z)),
                    jnp.sum(jnp.where(sel1, cosl, z)), jnp.sum(jnp.where(sel1, sinl, z)))

        bx1, by1, bx2, by2, bco, bsi = jax.lax.cond(
            cnt > 1.5, tie_fix, lambda _: g, None)

        bar = jnp.maximum(bx2 - bx1, 0.0) * jnp.maximum(by2 - by1, 0.0)
        xx1 = jnp.maximum(x1, bx1)
        yy1 = jnp.maximum(y1, by1)
        xx2 = jnp.minimum(x2, bx2)
        yy2 = jnp.minimum(y2, by2)
        inter = jnp.maximum(xx2 - xx1, 0.0) * jnp.maximum(yy2 - yy1, 0.0)
        union = area + bar - inter
        iou = inter / jnp.maximum(union, 1e-8)
        work = jnp.where(iou >= 0.5, _NEG, work)
        vf = (bval > _NEG * 0.5).astype(jnp.float32)
        vals = jnp.where(
            sub8 == 0, bx1,
            jnp.where(sub8 == 1, by1,
                      jnp.where(sub8 == 2, bx2,
                                jnp.where(sub8 == 3, by2,
                                          jnp.where(sub8 == 4, 1.0,
                                                    jnp.where(sub8 == 5, bval,
                                                              jnp.where(sub8 == 6, bco, bsi))))))) * vf
        acc = jnp.where(lane8 == i, vals, acc)
        return work, acc

    _, acc = jax.lax.fori_loop(0, _K, body, (work0, acc0))
    out_ref[...] = acc


def kernel(proposals, box_logits, label_logits, box_cos_logits, box_sin_logits):
    def pad(col):
        return jnp.pad(col, (0, _NPAD - _N)).reshape(_ROWS, 128)

    chans = [proposals[:, 0], proposals[:, 1], proposals[:, 2], proposals[:, 3],
             box_logits[:, 0], box_logits[:, 1], box_logits[:, 2], box_logits[:, 3],
             label_logits[:, 1], box_cos_logits, box_sin_logits]
    inp = jnp.stack([pad(c) for c in chans])
    out = pl.pallas_call(
        _nms_body,
        out_shape=jax.ShapeDtypeStruct((8, 128), jnp.float32),
    )(inp)
    return out[:, :_K].T


# speedup vs baseline: 17.3185x; 1.0801x over previous
import numpy as np
import jax
import jax.numpy as jnp
from jax.experimental import pallas as pl
from jax.experimental.pallas import tpu as pltpu

_N = 5000
_ROWS = 40
_NPAD = _ROWS * 128
_K = 100
_NEG = -1e30
_CLIP = float(np.float32(np.log(1333.0 / 16.0)))


def _nms_body(inp_ref, out_ref):
    px1 = inp_ref[0]
    py1 = inp_ref[1]
    px2 = inp_ref[2]
    py2 = inp_ref[3]
    tx = inp_ref[4] / 10.0
    ty = inp_ref[5] / 10.0
    tw = inp_ref[6] / 5.0
    th = inp_ref[7] / 5.0
    score = inp_ref[8]
    cosl = inp_ref[9]
    sinl = inp_ref[10]

    wa = px2 - px1
    ha = py2 - py1
    xa = (px2 + px1) * 0.5
    ya = (py2 + py1) * 0.5
    wb = jnp.exp(jnp.minimum(tw, _CLIP)) * wa
    hb = jnp.exp(jnp.minimum(th, _CLIP)) * ha
    xb = tx * wa + xa
    yb = ty * ha + ya
    x1 = jnp.clip(xb - wb * 0.5, 0.0, 1024.0)
    y1 = jnp.clip(yb - hb * 0.5, 0.0, 1024.0)
    x2 = jnp.clip(xb + wb * 0.5, 0.0, 1024.0)
    y2 = jnp.clip(yb + hb * 0.5, 0.0, 1024.0)

    area = jnp.maximum(x2 - x1, 0.0) * jnp.maximum(y2 - y1, 0.0)
    idx = (jax.lax.broadcasted_iota(jnp.int32, (_ROWS, 128), 0) * 128
           + jax.lax.broadcasted_iota(jnp.int32, (_ROWS, 128), 1))
    valid0 = (score > 0.05) & ((x2 - x1) * (y2 - y1) > 0.0) & (idx < _N)
    work0 = jnp.where(valid0, score, _NEG)

    sub8 = jax.lax.broadcasted_iota(jnp.int32, (8, 128), 0)
    lane8 = jax.lax.broadcasted_iota(jnp.int32, (8, 128), 1)
    acc0 = jnp.zeros((8, 128), jnp.float32)
    fidx = idx.astype(jnp.float32)

    ones128 = jnp.ones((128, 128), jnp.float32)
    z = jnp.float32(0.0)

    def body(i, state):
        work, acc = state

        bval = jnp.max(work)
        sel0 = work == bval

        def gather(sel):
            return [jnp.sum(jnp.where(sel, c, z))
                    for c in (x1, y1, x2, y2, cosl, sinl)] + [
                        jnp.sum(jnp.where(sel, 1.0, z))]

        def step(sel, work_in, acc_in):
            bx1, by1, bx2, by2, bco, bsi, cnt = gather(sel)
            bar = jnp.maximum(bx2 - bx1, 0.0) * jnp.maximum(by2 - by1, 0.0)
            xx1 = jnp.maximum(x1, bx1)
            yy1 = jnp.maximum(y1, by1)
            xx2 = jnp.minimum(x2, bx2)
            yy2 = jnp.minimum(y2, by2)
            inter = jnp.maximum(xx2 - xx1, 0.0) * jnp.maximum(yy2 - yy1, 0.0)
            union = area + bar - inter
            iou = inter / jnp.maximum(union, 1e-8)
            work_out = jnp.where(iou >= 0.5, _NEG, work_in)
            vf = (bval > _NEG * 0.5).astype(jnp.float32)
            vals = jnp.where(
                sub8 == 0, bx1,
                jnp.where(sub8 == 1, by1,
                          jnp.where(sub8 == 2, bx2,
                                    jnp.where(sub8 == 3, by2,
                                              jnp.where(sub8 == 4, 1.0,
                                                        jnp.where(sub8 == 5, bval,
                                                                  jnp.where(sub8 == 6, bco, bsi))))))) * vf
            acc_out = jnp.where(lane8 == i, vals, acc_in)
            return work_out, acc_out, cnt

        work1, acc1, cnt = step(sel0, work, acc)

        def tie_fix(_):
            bfi = jnp.min(jnp.where(sel0, fidx, jnp.float32(2 ** 30)))
            w2, a2, _unused = step(fidx == bfi, work, acc)
            return w2, a2

        work2, acc2 = jax.lax.cond(
            cnt > 1.5, tie_fix, lambda _: (work1, acc1), None)
        return work2, acc2

    _, acc = jax.lax.fori_loop(0, _K, body, (work0, acc0))
    out_ref[...] = acc


def kernel(proposals, box_logits, label_logits, box_cos_logits, box_sin_logits):
    x = jnp.concatenate([proposals, box_logits, label_logits[:, 1:2],
                         box_cos_logits[:, None], box_sin_logits[:, None]],
                        axis=1)
    x = jnp.pad(x, ((0, _NPAD - _N), (0, 0)))
    inp = x.T.reshape(11, _ROWS, 128)
    out = pl.pallas_call(
        _nms_body,
        out_shape=jax.ShapeDtypeStruct((8, 128), jnp.float32),
    )(inp)
    return out[:, :_K].T
